# baseline (speedup 1.0000x reference)
import jax
import jax.numpy as jnp
from jax.experimental import pallas as pl

_BR1 = 400
_BR = 1000


def _p1(a_ref, e0_ref, s1_ref, m1_ref, abf_ref):
    @pl.when(pl.program_id(0) == 0)
    def _():
        m1_ref[...] = jnp.zeros_like(m1_ref)

    a = a_ref[...].astype(jnp.bfloat16)
    abf_ref[...] = a
    w = (s1_ref[...] * e0_ref[...]).astype(jnp.bfloat16)
    m1_ref[...] += jax.lax.dot_general(
        a, w, (((0,), (0,)), ((), ())), preferred_element_type=jnp.float32)


def _p2(a_ref, m1_ref, e_ref, d_ref, s2_ref, e1_ref, m2_ref):
    @pl.when(pl.program_id(0) == 0)
    def _():
        m2_ref[...] = jnp.zeros_like(m2_ref)

    a = a_ref[...]
    y1 = (e_ref[...] * m1_ref[...]).astype(jnp.bfloat16)
    t = jax.lax.dot_general(
        a, y1, (((1,), (0,)), ((), ())), preferred_element_type=jnp.float32)
    e1_ref[...] = d_ref[...] * t
    m2_ref[...] += jax.lax.dot_general(
        a, (s2_ref[...] * t).astype(jnp.bfloat16), (((0,), (0,)), ((), ())),
        preferred_element_type=jnp.float32)


def _p3(a_ref, m2_ref, e_ref, d_ref, e0_ref, e1_ref, out_ref):
    a = a_ref[...]
    y2 = (e_ref[...] * m2_ref[...]).astype(jnp.bfloat16)
    t = jax.lax.dot_general(
        a, y2, (((1,), (0,)), ((), ())), preferred_element_type=jnp.float32)
    out_ref[...] = (e0_ref[...] + e1_ref[...] + d_ref[...] * t) * (1.0 / 3.0)


def kernel(users_embedding, product_embedding, adj_matrix, degreeV_matrix,
           degreeE_matrix, gate_user, gate_product):
    num_users, dim = users_embedding.shape
    n = num_users + product_embedding.shape[0]
    b = adj_matrix.shape[1]
    assert n % _BR == 0 and n % _BR1 == 0
    nsteps1 = n // _BR1
    nsteps = n // _BR

    e0 = jnp.concatenate([users_embedding, product_embedding], axis=0)
    g = jnp.where(jnp.arange(n) < num_users, gate_user, gate_product)
    d = degreeV_matrix
    s1 = (d * g)[:, None].astype(jnp.float32)
    s2 = (d * d * g)[:, None].astype(jnp.float32)
    dcol = d[:, None]
    e = degreeE_matrix[:, None]

    m1, abf = pl.pallas_call(
        _p1,
        grid=(nsteps1,),
        in_specs=[
            pl.BlockSpec((_BR1, b), lambda k: (k, 0)),
            pl.BlockSpec((_BR1, dim), lambda k: (k, 0)),
            pl.BlockSpec((_BR1, 1), lambda k: (k, 0)),
        ],
        out_specs=[
            pl.BlockSpec((b, dim), lambda k: (0, 0)),
            pl.BlockSpec((_BR1, b), lambda k: (k, 0)),
        ],
        out_shape=[
            jax.ShapeDtypeStruct((b, dim), jnp.float32),
            jax.ShapeDtypeStruct((n, b), jnp.bfloat16),
        ],
    )(adj_matrix, e0, s1)

    e1, m2 = pl.pallas_call(
        _p2,
        grid=(nsteps,),
        in_specs=[
            pl.BlockSpec((_BR, b), lambda k: (k, 0)),
            pl.BlockSpec((b, dim), lambda k: (0, 0)),
            pl.BlockSpec((b, 1), lambda k: (0, 0)),
            pl.BlockSpec((_BR, 1), lambda k: (k, 0)),
            pl.BlockSpec((_BR, 1), lambda k: (k, 0)),
        ],
        out_specs=[
            pl.BlockSpec((_BR, dim), lambda k: (k, 0)),
            pl.BlockSpec((b, dim), lambda k: (0, 0)),
        ],
        out_shape=[
            jax.ShapeDtypeStruct((n, dim), jnp.float32),
            jax.ShapeDtypeStruct((b, dim), jnp.float32),
        ],
    )(abf, m1, e, dcol, s2)

    out = pl.pallas_call(
        _p3,
        grid=(nsteps,),
        in_specs=[
            pl.BlockSpec((_BR, b), lambda k: (k, 0)),
            pl.BlockSpec((b, dim), lambda k: (0, 0)),
            pl.BlockSpec((b, 1), lambda k: (0, 0)),
            pl.BlockSpec((_BR, 1), lambda k: (k, 0)),
            pl.BlockSpec((_BR, dim), lambda k: (k, 0)),
            pl.BlockSpec((_BR, dim), lambda k: (k, 0)),
        ],
        out_specs=pl.BlockSpec((_BR, dim), lambda k: (k, 0)),
        out_shape=jax.ShapeDtypeStruct((n, dim), jnp.float32),
    )(abf, m2, e, dcol, e0, e1)

    return (out[:num_users], out[num_users:])


# speedup vs baseline: 1.1653x; 1.1653x over previous
import jax
import jax.numpy as jnp
from jax.experimental import pallas as pl
from jax.experimental.pallas import tpu as pltpu

_BR1 = 400
_BR = 1000


def _p1(u_ref, p_ref, a_ref, d_ref, gu_ref, gp_ref, m1t_ref, abf_ref, *, nu_blocks):
    k = pl.program_id(0)

    @pl.when(k == 0)
    def _():
        m1t_ref[...] = jnp.zeros_like(m1t_ref)

    a = a_ref[...].astype(jnp.bfloat16)
    abf_ref[...] = a
    is_user = k < nu_blocks
    e0 = jnp.where(is_user, u_ref[...], p_ref[...])
    g = jnp.where(is_user, gu_ref[0, 0], gp_ref[0, 0])
    w = (g * d_ref[...] * e0).astype(jnp.bfloat16)
    m1t_ref[...] += jax.lax.dot_general(
        w, a, (((0,), (0,)), ((), ())), preferred_element_type=jnp.float32)


def _p2(a_ref, m1t_ref, e_ref, d_ref, gu_ref, gp_ref, e1_ref, m2t_ref, y1_scr,
        *, nu_blocks):
    k = pl.program_id(0)

    @pl.when(k == 0)
    def _():
        m2t_ref[...] = jnp.zeros_like(m2t_ref)
        y1_scr[...] = (e_ref[...] * m1t_ref[...]).astype(jnp.bfloat16).T

    a = a_ref[...]
    t = jax.lax.dot_general(
        a, y1_scr[...], (((1,), (0,)), ((), ())),
        preferred_element_type=jnp.float32)
    d = d_ref[...]
    e1_ref[...] = d * t
    g = jnp.where(k < nu_blocks, gu_ref[0, 0], gp_ref[0, 0])
    x = (g * d * d * t).astype(jnp.bfloat16)
    m2t_ref[...] += jax.lax.dot_general(
        x, a, (((0,), (0,)), ((), ())), preferred_element_type=jnp.float32)


def _p3(a_ref, m2t_ref, e_ref, d_ref, u_ref, p_ref, e1_ref, uo_ref, po_ref,
        y2_scr, *, nu_blocks):
    k = pl.program_id(0)

    @pl.when(k == 0)
    def _():
        y2_scr[...] = (e_ref[...] * m2t_ref[...]).astype(jnp.bfloat16).T

    a = a_ref[...]
    t = jax.lax.dot_general(
        a, y2_scr[...], (((1,), (0,)), ((), ())),
        preferred_element_type=jnp.float32)
    is_user = k < nu_blocks
    e0 = jnp.where(is_user, u_ref[...], p_ref[...])
    res = (e0 + e1_ref[...] + d_ref[...] * t) * (1.0 / 3.0)

    @pl.when(is_user)
    def _():
        uo_ref[...] = res

    @pl.when(jnp.logical_not(is_user))
    def _():
        po_ref[...] = res


def kernel(users_embedding, product_embedding, adj_matrix, degreeV_matrix,
           degreeE_matrix, gate_user, gate_product):
    import functools

    nu, dim = users_embedding.shape
    npr = product_embedding.shape[0]
    n = nu + npr
    b = adj_matrix.shape[1]
    assert nu % _BR1 == 0 and npr % _BR1 == 0 and nu % _BR == 0 and npr % _BR == 0
    nsteps1 = n // _BR1
    nub1 = nu // _BR1
    nsteps = n // _BR
    nub = nu // _BR

    dcol = degreeV_matrix[:, None]
    erow = degreeE_matrix[None, :]
    gu = gate_user.reshape(1, 1)
    gp = gate_product.reshape(1, 1)

    m1t, abf = pl.pallas_call(
        functools.partial(_p1, nu_blocks=nub1),
        grid=(nsteps1,),
        in_specs=[
            pl.BlockSpec((_BR1, dim), lambda k: (jnp.minimum(k, nub1 - 1), 0)),
            pl.BlockSpec((_BR1, dim),
                         lambda k: (jnp.maximum(k - nub1, 0), 0)),
            pl.BlockSpec((_BR1, b), lambda k: (k, 0)),
            pl.BlockSpec((_BR1, 1), lambda k: (k, 0)),
            pl.BlockSpec((1, 1), lambda k: (0, 0)),
            pl.BlockSpec((1, 1), lambda k: (0, 0)),
        ],
        out_specs=[
            pl.BlockSpec((dim, b), lambda k: (0, 0)),
            pl.BlockSpec((_BR1, b), lambda k: (k, 0)),
        ],
        out_shape=[
            jax.ShapeDtypeStruct((dim, b), jnp.float32),
            jax.ShapeDtypeStruct((n, b), jnp.bfloat16),
        ],
    )(users_embedding, product_embedding, adj_matrix, dcol, gu, gp)

    e1, m2t = pl.pallas_call(
        functools.partial(_p2, nu_blocks=nub),
        grid=(nsteps,),
        in_specs=[
            pl.BlockSpec((_BR, b), lambda k: (k, 0)),
            pl.BlockSpec((dim, b), lambda k: (0, 0)),
            pl.BlockSpec((1, b), lambda k: (0, 0)),
            pl.BlockSpec((_BR, 1), lambda k: (k, 0)),
            pl.BlockSpec((1, 1), lambda k: (0, 0)),
            pl.BlockSpec((1, 1), lambda k: (0, 0)),
        ],
        out_specs=[
            pl.BlockSpec((_BR, dim), lambda k: (k, 0)),
            pl.BlockSpec((dim, b), lambda k: (0, 0)),
        ],
        out_shape=[
            jax.ShapeDtypeStruct((n, dim), jnp.float32),
            jax.ShapeDtypeStruct((dim, b), jnp.float32),
        ],
        scratch_shapes=[pltpu.VMEM((b, dim), jnp.bfloat16)],
    )(abf, m1t, erow, dcol, gu, gp)

    user_emb, product_emb = pl.pallas_call(
        functools.partial(_p3, nu_blocks=nub),
        grid=(nsteps,),
        in_specs=[
            pl.BlockSpec((_BR, b), lambda k: (k, 0)),
            pl.BlockSpec((dim, b), lambda k: (0, 0)),
            pl.BlockSpec((1, b), lambda k: (0, 0)),
            pl.BlockSpec((_BR, 1), lambda k: (k, 0)),
            pl.BlockSpec((_BR, dim), lambda k: (jnp.minimum(k, nub - 1), 0)),
            pl.BlockSpec((_BR, dim), lambda k: (jnp.maximum(k - nub, 0), 0)),
            pl.BlockSpec((_BR, dim), lambda k: (k, 0)),
        ],
        out_specs=[
            pl.BlockSpec((_BR, dim), lambda k: (jnp.minimum(k, nub - 1), 0)),
            pl.BlockSpec((_BR, dim), lambda k: (jnp.maximum(k - nub, 0), 0)),
        ],
        out_shape=[
            jax.ShapeDtypeStruct((nu, dim), jnp.float32),
            jax.ShapeDtypeStruct((npr, dim), jnp.float32),
        ],
        scratch_shapes=[pltpu.VMEM((b, dim), jnp.bfloat16)],
    )(abf, m2t, erow, dcol, users_embedding, product_embedding, e1)

    return (user_emb, product_emb)
